# baseline (speedup 1.0000x reference)
import functools

import jax
import jax.numpy as jnp
from jax import lax
from jax.experimental import pallas as pl
from jax.experimental.pallas import tpu as pltpu
from jax.experimental.pallas import tpu_sc as plsc

_B = 64
_S = 50
_T = _S - 2
_E = 32
_C = 32
_NB = 16
_NC, _NS = 2, 16
_NW = _NC * _NS
_PAIRS = _B * _T
_PPW = _PAIRS // _NW
_CHUNK = 128
_NCHUNK = _PPW * _NB // _CHUNK
_RC = 384
_NRC = _PAIRS // _RC


_NBPACK = 50176


def _pack_body(qq_r, ql_r, ll_r, lq_r, out_r):
    out_r[:, pl.ds(0, _NB)] = qq_r[...]
    out_r[:, pl.ds(_NB, _NB)] = ql_r[...]
    out_r[:, pl.ds(2 * _NB, _NB)] = ll_r[...]
    out_r[:, pl.ds(3 * _NB, _NB)] = lq_r[...]


@functools.cache
def _pack_fn():
    blk = 1024
    spec = pl.BlockSpec((blk, _NB), lambda i: (i, 0))
    return pl.pallas_call(
        _pack_body,
        grid=(_NBPACK // blk,),
        in_specs=[spec] * 4,
        out_specs=pl.BlockSpec((blk, 128), lambda i: (i, 0)),
        out_shape=jax.ShapeDtypeStruct((_NBPACK, 128), jnp.int32),
    )


def _sc_body(qi_hbm, li_hbm, nb_hbm, qe_hbm, le_hbm,
             out_qle, out_x4,
             idx_v, nbr_v, flat_a, flat_b, rows_v, erow_v, ebuf_v, sem, esem):
    wid = lax.axis_index("s") * _NC + lax.axis_index("c")
    base = wid * _PPW
    nflat = _PPW * _NB
    nch = nflat // _CHUNK

    def side(i_hbm, e_hbm, nbcol, emb_a, emb_b, qle_col, x4_col):
        pltpu.sync_copy(i_hbm.at[pl.ds(base, _PPW)], idx_v)
        ecp = pltpu.async_copy(e_hbm.at[idx_v], ebuf_v, esem)
        pltpu.async_copy(nb_hbm.at[idx_v], nbr_v, sem).wait()

        def flat_body(p, carry):
            flat_a[pl.ds(p * _NB, _NB)] = nbr_v[p, pl.ds(nbcol, _NB)]
            flat_b[pl.ds(p * _NB, _NB)] = nbr_v[p, pl.ds(nbcol + _NB, _NB)]
            return carry

        lax.fori_loop(0, _PPW, flat_body, 0)
        ecp.wait()
        pltpu.sync_copy(ebuf_v, out_qle.at[pl.ds(base, _PPW), pl.ds(qle_col, _E)])

        for half, (flat, emb) in enumerate(((flat_a, emb_a), (flat_b, emb_b))):
            copies = [
                pltpu.async_copy(emb.at[flat.at[pl.ds(c * _CHUNK, _CHUNK)]],
                                 rows_v.at[pl.ds(c * _CHUNK, _CHUNK)], sem)
                for c in range(nch)
            ]
            for cp in copies:
                cp.wait()

            col0 = half * _E

            def acc_body(p, carry):
                lo = rows_v[p * _NB, pl.ds(0, 16)]
                hi = rows_v[p * _NB, pl.ds(16, 16)]
                for j in range(1, _NB):
                    lo = lo + rows_v[p * _NB + j, pl.ds(0, 16)]
                    hi = hi + rows_v[p * _NB + j, pl.ds(16, 16)]
                nv = nbr_v[p, pl.ds(nbcol + half * _NB, _NB)]
                cnt = jnp.int32(0)
                for j in range(_NB):
                    cnt = cnt + jnp.where(nv[j] != 0, 1, 0).astype(jnp.int32)
                inv = jnp.float32(1.0)
                for k in range(2, _NB + 1):
                    inv = jnp.where(cnt == k, jnp.float32(1.0 / k), inv)
                erow_v[p, pl.ds(col0, 16)] = lo * inv
                erow_v[p, pl.ds(col0 + 16, 16)] = hi * inv
                return carry

            lax.fori_loop(0, _PPW, acc_body, 0)

        pltpu.sync_copy(
            erow_v, out_x4.at[pl.ds(base, _PPW), pl.ds(x4_col, 2 * _E)])

    side(qi_hbm, qe_hbm, 0, qe_hbm, le_hbm, 0, 0)
    side(li_hbm, le_hbm, 2 * _NB, le_hbm, qe_hbm, _E, 2 * _E)


@functools.cache
def _sc_gather_fn():
    return pl.kernel(
        _sc_body,
        out_type=[jax.ShapeDtypeStruct((_PAIRS, 2 * _E), jnp.float32),
                  jax.ShapeDtypeStruct((_PAIRS, 4 * _E), jnp.float32)],
        mesh=plsc.VectorSubcoreMesh(core_axis_name="c", subcore_axis_name="s",
                                    num_cores=_NC, num_subcores=_NS),
        compiler_params=pltpu.CompilerParams(use_tc_tiling_on_sc=False),
        scratch_types=[
            pltpu.VMEM((_PPW,), jnp.int32),
            pltpu.VMEM((_PPW, 128), jnp.int32),
            pltpu.VMEM((_PPW * _NB,), jnp.int32),
            pltpu.VMEM((_PPW * _NB,), jnp.int32),
            pltpu.VMEM((_PPW * _NB, _E), jnp.float32),
            pltpu.VMEM((_PPW, 2 * _E), jnp.float32),
            pltpu.VMEM((_PPW, _E), jnp.float32),
            pltpu.SemaphoreType.DMA,
            pltpu.SemaphoreType.DMA,
        ],
    )


def _sig(x):
    return 1.0 / (1.0 + jnp.exp(-x))


def _tc_body(x4_r, qle_r, aux_r, aemb_r,
             wnb_r, wg2_r, bg2_r, wk2_r, bk2_r, keyt_r,
             weaq_r, beaq_r, weal_r, beal_r,
             tcat_r, wsq_r, bsum_r, wsr_r, wout_r, wtq_r, wtl_r, vmi_r,
             bout_r, btq_r, btl_r,
             out_r,
             vm_r, w_s, er_s, ad_s, ps_s, rd_s):
    Wnb = wnb_r[...]; Wg2 = wg2_r[...]; bg2 = bg2_r[...]
    Wk2 = wk2_r[...]; bk2 = bk2_r[...]; keyT = keyt_r[...]
    WeaQ = weaq_r[...]; beaQ = beaq_r[...]
    WeaL = weal_r[...]; beaL = beal_r[...]
    Tcat = tcat_r[...]; Wsq = wsq_r[...]; bsum = bsum_r[...]; Wsr = wsr_r[...]
    wout = wout_r[...]; wtq = wtq_r[...]; wtl = wtl_r[...]
    bout = bout_r[0]; btq = btq_r[0]; btl = btl_r[0]
    a0 = aemb_r[0:1, :]
    a1 = aemb_r[1:2, :]

    def phase_a(i, carry):
        b0 = i * _RC
        x4 = x4_r[pl.ds(b0, _RC), :]
        qle = qle_r[pl.ds(b0, _RC), :]
        aux = aux_r[pl.ds(b0, _RC), :]
        iqc = aux[:, 0:1] > 0.5
        am = aux[:, 1:2]
        a_t = am * a1 + (1.0 - am) * a0
        y = jnp.dot(x4, Wnb)
        z = jnp.tanh(jnp.dot(qle + y, Wg2) + bg2)
        u = jnp.dot(z, Wk2) + bk2
        usel = jnp.where(iqc, u[:, :_E], u[:, _E:])
        klog = jnp.dot(usel, keyT)
        mx = jnp.max(klog, axis=1, keepdims=True)
        ex = jnp.exp(klog - mx)
        w = ex / jnp.sum(ex, axis=1, keepdims=True)
        q_t = z[:, :_E]
        eaq = jnp.dot(jnp.concatenate([q_t, a_t], axis=1), WeaQ) + beaQ
        eal = jnp.dot(z[:, _E:], WeaL) + beaL
        er = jnp.where(iqc, _sig(eaq[:, :_E]), _sig(eal[:, :_E]))
        ad = jnp.where(iqc, jnp.tanh(eaq[:, _E:]), jnp.tanh(eal[:, _E:]))
        ps = jnp.dot(q_t, Wsq) + bsum
        w_s[pl.ds(b0, _RC), :] = w
        er_s[pl.ds(b0, _RC), :] = er
        ad_s[pl.ds(b0, _RC), :] = ad
        ps_s[pl.ds(b0, _RC), :] = ps
        return carry

    lax.fori_loop(0, _NRC, phase_a, 0, unroll=False)

    vm_r[...] = jnp.broadcast_to(vmi_r[...], (_B, _C, _E))

    def phase_b(t, carry):
        b0 = t * _B
        aux = aux_r[pl.ds(b0, _B), :]
        iq3 = aux[:, 0:1][:, :, None] > 0.5
        pq3 = aux[:, 2:3][:, :, None] > 0.5
        vm2 = vm_r[...].reshape(_B * _C, _E)
        prod = jnp.dot(vm2, Tcat)
        vqq = prod[:, 0:_E].reshape(_B, _C, _E)
        vql = prod[:, _E:2 * _E].reshape(_B, _C, _E)
        vlq = prod[:, 2 * _E:3 * _E].reshape(_B, _C, _E)
        vll = prod[:, 3 * _E:].reshape(_B, _C, _E)
        vm = jnp.where(pq3, jnp.where(iq3, vqq, vql), jnp.where(iq3, vlq, vll))
        w3 = w_s[pl.ds(b0, _B), :][:, :, None]
        read = jnp.sum(w3 * vm, axis=1)
        rd_s[pl.ds(b0, _B), :] = read
        er = er_s[pl.ds(b0, _B), :]
        ad = ad_s[pl.ds(b0, _B), :]
        vm_r[...] = vm * (1.0 - w3 * er[:, None, :]) + w3 * ad[:, None, :]
        return carry

    lax.fori_loop(0, _T, phase_b, 0, unroll=False)

    def phase_c(i, carry):
        b0 = i * _RC
        read = rd_s[pl.ds(b0, _RC), :]
        aux = aux_r[pl.ds(b0, _RC), :]
        summ = jnp.tanh(jnp.dot(read, Wsr) + ps_s[pl.ds(b0, _RC), :])
        pred = _sig(jnp.sum(summ * wout, axis=1, keepdims=True) + bout)
        ptq = jnp.sum(read * wtq, axis=1, keepdims=True) + btq
        ptl = jnp.sum(read * wtl, axis=1, keepdims=True) + btl
        ptype = _sig(jnp.where(aux[:, 0:1] > 0.5, ptq, ptl))
        out_r[pl.ds(b0, _RC), :] = jnp.concatenate([pred, ptype], axis=1)
        return carry

    lax.fori_loop(0, _NRC, phase_c, 0, unroll=False)


_N_VMEM_IN = 22


@functools.cache
def _tc_scan_fn():
    return pl.pallas_call(
        _tc_body,
        out_shape=[jax.ShapeDtypeStruct((_PAIRS, 2), jnp.float32)],
        in_specs=[pl.BlockSpec(memory_space=pltpu.VMEM)] * _N_VMEM_IN
        + [pl.BlockSpec(memory_space=pltpu.SMEM)] * 3,
        scratch_shapes=[
            pltpu.VMEM((_B, _C, _E), jnp.float32),
            pltpu.VMEM((_PAIRS, _E), jnp.float32),
            pltpu.VMEM((_PAIRS, _E), jnp.float32),
            pltpu.VMEM((_PAIRS, _E), jnp.float32),
            pltpu.VMEM((_PAIRS, 2 * _E), jnp.float32),
            pltpu.VMEM((_PAIRS, _E), jnp.float32),
        ],
    )


def kernel(q_data, a_data, l_data, d_data, q_q_neighbors, q_l_neighbors,
           l_q_neighbors, l_l_neighbors, q_embed, l_embed, a_embed,
           key_matrix, value_matrix_init,
           W_QQ, W_QL, W_LL, W_LQ, W_GQ, b_GQ, W_GL, b_GL,
           W_kq, b_kq, W_kl, b_kl, W_eq, b_eq, W_el, b_el,
           W_aq, b_aq, W_al, b_al, T_QQ, T_QL, T_LQ, T_LL,
           W_sum, b_sum, W_out, b_out, W_tq, b_tq, W_tl, b_tl):
    f32 = jnp.float32
    qi = q_data[:, 1:_S - 1].T.reshape(-1).astype(jnp.int32)
    li = l_data[:, 1:_S - 1].T.reshape(-1).astype(jnp.int32)
    aux = jnp.stack([
        (d_data[:, 1:_S - 1].T == 0).astype(f32),
        a_data[:, 1:_S - 1].T.astype(f32),
        (d_data[:, 0:_S - 2].T == 0).astype(f32),
        jnp.zeros((_T, _B), f32),
    ], axis=-1).reshape(_PAIRS, 4)

    nball = _pack_fn()(q_q_neighbors, q_l_neighbors,
                       l_l_neighbors, l_q_neighbors)
    qle, x4 = _sc_gather_fn()(qi, li, nball, q_embed, l_embed)

    z32 = jnp.zeros((_E, _E), f32)
    Wnb = jnp.concatenate([
        jnp.concatenate([W_QQ, z32], axis=1),
        jnp.concatenate([W_QL, z32], axis=1),
        jnp.concatenate([z32, W_LL], axis=1),
        jnp.concatenate([z32, W_LQ], axis=1),
    ], axis=0)
    bd = lambda a, b: jnp.concatenate([
        jnp.concatenate([a, z32], axis=1),
        jnp.concatenate([z32, b], axis=1)], axis=0)
    cat1 = lambda a, b: jnp.concatenate([a, b], axis=1)
    Wg2 = bd(W_GQ, W_GL)
    bg2 = cat1(b_GQ.reshape(1, _E), b_GL.reshape(1, _E))
    Wk2 = bd(W_kq, W_kl)
    bk2 = cat1(b_kq.reshape(1, _E), b_kl.reshape(1, _E))
    WeaQ = jnp.concatenate([cat1(W_eq[:_E], W_aq[:_E]),
                            cat1(W_eq[_E:], W_aq[_E:])], axis=0)
    beaQ = cat1(b_eq.reshape(1, _E), b_aq.reshape(1, _E))
    WeaL = cat1(W_el, W_al)
    beaL = cat1(b_el.reshape(1, _E), b_al.reshape(1, _E))
    Tcat = jnp.concatenate([T_QQ, T_QL, T_LQ, T_LL], axis=1)

    (out,) = _tc_scan_fn()(
        x4, qle, aux, a_embed,
        Wnb, Wg2, bg2, Wk2, bk2, key_matrix.T,
        WeaQ, beaQ, WeaL, beaL,
        Tcat, W_sum[_E:], b_sum.reshape(1, -1), W_sum[:_E],
        W_out.T, W_tq.T, W_tl.T, value_matrix_init,
        b_out, b_tq, b_tl)

    return jnp.swapaxes(out.reshape(_T, _B, 2), 0, 1)


# speedup vs baseline: 1.2556x; 1.2556x over previous
import functools

import jax
import jax.numpy as jnp
from jax import lax
from jax.experimental import pallas as pl
from jax.experimental.pallas import tpu as pltpu
from jax.experimental.pallas import tpu_sc as plsc

_B = 64
_S = 50
_T = _S - 2
_E = 32
_C = 32
_NB = 16
_NC, _NS = 2, 16
_NW = _NC * _NS
_PAIRS = _B * _T
_PPW = _PAIRS // _NW
_CHUNK = 128
_NCHUNK = _PPW * _NB // _CHUNK
_RC = 384
_NRC = _PAIRS // _RC


def _sc_body(qi_hbm, li_hbm, nbq_hbm, nbl_hbm, qe_hbm, le_hbm,
             out_qle, out_x4,
             idx_v, nbr_v, flat_a, flat_b, rows_v, erow_v, ebuf_v, sem, esem):
    wid = lax.axis_index("s") * _NC + lax.axis_index("c")
    base = wid * _PPW
    nflat = _PPW * _NB
    nch = nflat // _CHUNK

    def side(i_hbm, e_hbm, nb_hbm, emb_a, emb_b, qle_col, x4_col):
        pltpu.sync_copy(i_hbm.at[pl.ds(base, _PPW)], idx_v)
        ecp = pltpu.async_copy(e_hbm.at[idx_v], ebuf_v, esem)
        pltpu.async_copy(nb_hbm.at[idx_v], nbr_v, sem).wait()

        def flat_body(p, carry):
            flat_a[pl.ds(p * _NB, _NB)] = nbr_v[p, pl.ds(0, _NB)]
            flat_b[pl.ds(p * _NB, _NB)] = nbr_v[p, pl.ds(_NB, _NB)]
            return carry

        lax.fori_loop(0, _PPW, flat_body, 0)
        ecp.wait()
        pltpu.sync_copy(ebuf_v, out_qle.at[pl.ds(base, _PPW), pl.ds(qle_col, _E)])

        for half, (flat, emb) in enumerate(((flat_a, emb_a), (flat_b, emb_b))):
            copies = [
                pltpu.async_copy(emb.at[flat.at[pl.ds(c * _CHUNK, _CHUNK)]],
                                 rows_v.at[pl.ds(c * _CHUNK, _CHUNK)], sem)
                for c in range(nch)
            ]
            for cp in copies:
                cp.wait()

            col0 = half * _E

            def acc_body(p, carry):
                lo = rows_v[p * _NB, pl.ds(0, 16)]
                hi = rows_v[p * _NB, pl.ds(16, 16)]
                for j in range(1, _NB):
                    lo = lo + rows_v[p * _NB + j, pl.ds(0, 16)]
                    hi = hi + rows_v[p * _NB + j, pl.ds(16, 16)]
                nv = nbr_v[p, pl.ds(half * _NB, _NB)]
                cnt = jnp.int32(0)
                for j in range(_NB):
                    cnt = cnt + jnp.where(nv[j] != 0, 1, 0).astype(jnp.int32)
                inv = jnp.float32(1.0)
                for k in range(2, _NB + 1):
                    inv = jnp.where(cnt == k, jnp.float32(1.0 / k), inv)
                erow_v[p, pl.ds(col0, 16)] = lo * inv
                erow_v[p, pl.ds(col0 + 16, 16)] = hi * inv
                return carry

            lax.fori_loop(0, _PPW, acc_body, 0)

        pltpu.sync_copy(
            erow_v, out_x4.at[pl.ds(base, _PPW), pl.ds(x4_col, 2 * _E)])

    side(qi_hbm, qe_hbm, nbq_hbm, qe_hbm, le_hbm, 0, 0)
    side(li_hbm, le_hbm, nbl_hbm, le_hbm, qe_hbm, _E, 2 * _E)


@functools.cache
def _sc_gather_fn():
    return pl.kernel(
        _sc_body,
        out_type=[jax.ShapeDtypeStruct((_PAIRS, 2 * _E), jnp.float32),
                  jax.ShapeDtypeStruct((_PAIRS, 4 * _E), jnp.float32)],
        mesh=plsc.VectorSubcoreMesh(core_axis_name="c", subcore_axis_name="s",
                                    num_cores=_NC, num_subcores=_NS),
        compiler_params=pltpu.CompilerParams(use_tc_tiling_on_sc=False),
        scratch_types=[
            pltpu.VMEM((_PPW,), jnp.int32),
            pltpu.VMEM((_PPW, 2 * _NB), jnp.int32),
            pltpu.VMEM((_PPW * _NB,), jnp.int32),
            pltpu.VMEM((_PPW * _NB,), jnp.int32),
            pltpu.VMEM((_PPW * _NB, _E), jnp.float32),
            pltpu.VMEM((_PPW, 2 * _E), jnp.float32),
            pltpu.VMEM((_PPW, _E), jnp.float32),
            pltpu.SemaphoreType.DMA,
            pltpu.SemaphoreType.DMA,
        ],
    )


def _sig(x):
    return 1.0 / (1.0 + jnp.exp(-x))


def _tc_body(x4_r, qle_r, aux_r, aemb_r,
             wnb_r, wg2_r, bg2_r, wk2_r, bk2_r, keyt_r,
             weaq_r, beaq_r, weal_r, beal_r,
             tcat_r, wsq_r, bsum_r, wsr_r, wout_r, wtq_r, wtl_r, vmi_r,
             bout_r, btq_r, btl_r,
             out_r,
             w_s, er_s, ad_s, ps_s, rd_s):
    Wnb = wnb_r[...]; Wg2 = wg2_r[...]; bg2 = bg2_r[...]
    Wk2 = wk2_r[...]; bk2 = bk2_r[...]; keyT = keyt_r[...]
    WeaQ = weaq_r[...]; beaQ = beaq_r[...]
    WeaL = weal_r[...]; beaL = beal_r[...]
    Tcat = tcat_r[...]; Wsq = wsq_r[...]; bsum = bsum_r[...]; Wsr = wsr_r[...]
    wout = wout_r[...]; wtq = wtq_r[...]; wtl = wtl_r[...]
    bout = bout_r[0]; btq = btq_r[0]; btl = btl_r[0]
    a0 = aemb_r[0:1, :]
    a1 = aemb_r[1:2, :]

    def phase_a(i, carry):
        b0 = i * _RC
        x4 = x4_r[pl.ds(b0, _RC), :]
        qle = qle_r[pl.ds(b0, _RC), :]
        aux = aux_r[pl.ds(b0, _RC), :]
        iqc = aux[:, 0:1] > 0.5
        am = aux[:, 1:2]
        a_t = am * a1 + (1.0 - am) * a0
        y = jnp.dot(x4, Wnb)
        z = jnp.tanh(jnp.dot(qle + y, Wg2) + bg2)
        u = jnp.dot(z, Wk2) + bk2
        usel = jnp.where(iqc, u[:, :_E], u[:, _E:])
        klog = jnp.dot(usel, keyT)
        mx = jnp.max(klog, axis=1, keepdims=True)
        ex = jnp.exp(klog - mx)
        w = ex / jnp.sum(ex, axis=1, keepdims=True)
        q_t = z[:, :_E]
        eaq = jnp.dot(jnp.concatenate([q_t, a_t], axis=1), WeaQ) + beaQ
        eal = jnp.dot(z[:, _E:], WeaL) + beaL
        er = jnp.where(iqc, _sig(eaq[:, :_E]), _sig(eal[:, :_E]))
        ad = jnp.where(iqc, jnp.tanh(eaq[:, _E:]), jnp.tanh(eal[:, _E:]))
        ps = jnp.dot(q_t, Wsq) + bsum
        w_s[pl.ds(b0, _RC), :] = w
        er_s[pl.ds(b0, _RC), :] = er
        ad_s[pl.ds(b0, _RC), :] = ad
        ps_s[pl.ds(b0, _RC), :] = ps
        return carry

    lax.fori_loop(0, _NRC, phase_a, 0, unroll=False)

    def phase_b(t, vm):
        b0 = t * _B
        aux = aux_r[pl.ds(b0, _B), :]
        iq3 = aux[:, 0:1][:, :, None] > 0.5
        pq3 = aux[:, 2:3][:, :, None] > 0.5
        vm2 = vm.reshape(_B * _C, _E).astype(jnp.bfloat16)
        prod = jnp.dot(vm2, Tcat, preferred_element_type=jnp.float32)
        vqq = prod[:, 0:_E].reshape(_B, _C, _E)
        vql = prod[:, _E:2 * _E].reshape(_B, _C, _E)
        vlq = prod[:, 2 * _E:3 * _E].reshape(_B, _C, _E)
        vll = prod[:, 3 * _E:].reshape(_B, _C, _E)
        vm = jnp.where(pq3, jnp.where(iq3, vqq, vql), jnp.where(iq3, vlq, vll))
        w3 = w_s[pl.ds(b0, _B), :][:, :, None]
        read = jnp.sum(w3 * vm, axis=1)
        rd_s[pl.ds(b0, _B), :] = read
        er = er_s[pl.ds(b0, _B), :]
        ad = ad_s[pl.ds(b0, _B), :]
        return vm + w3 * (ad[:, None, :] - vm * er[:, None, :])

    lax.fori_loop(0, _T, phase_b,
                  jnp.broadcast_to(vmi_r[...], (_B, _C, _E)), unroll=False)

    def phase_c(i, carry):
        b0 = i * _RC
        read = rd_s[pl.ds(b0, _RC), :]
        aux = aux_r[pl.ds(b0, _RC), :]
        summ = jnp.tanh(jnp.dot(read, Wsr) + ps_s[pl.ds(b0, _RC), :])
        pred = _sig(jnp.sum(summ * wout, axis=1, keepdims=True) + bout)
        ptq = jnp.sum(read * wtq, axis=1, keepdims=True) + btq
        ptl = jnp.sum(read * wtl, axis=1, keepdims=True) + btl
        ptype = _sig(jnp.where(aux[:, 0:1] > 0.5, ptq, ptl))
        out_r[pl.ds(b0, _RC), :] = jnp.concatenate([pred, ptype], axis=1)
        return carry

    lax.fori_loop(0, _NRC, phase_c, 0, unroll=False)


_N_VMEM_IN = 22


@functools.cache
def _tc_scan_fn():
    return pl.pallas_call(
        _tc_body,
        out_shape=[jax.ShapeDtypeStruct((_PAIRS, 2), jnp.float32)],
        in_specs=[pl.BlockSpec(memory_space=pltpu.VMEM)] * _N_VMEM_IN
        + [pl.BlockSpec(memory_space=pltpu.SMEM)] * 3,
        scratch_shapes=[
            pltpu.VMEM((_PAIRS, _E), jnp.float32),
            pltpu.VMEM((_PAIRS, _E), jnp.float32),
            pltpu.VMEM((_PAIRS, _E), jnp.float32),
            pltpu.VMEM((_PAIRS, 2 * _E), jnp.float32),
            pltpu.VMEM((_PAIRS, _E), jnp.float32),
        ],
    )


def kernel(q_data, a_data, l_data, d_data, q_q_neighbors, q_l_neighbors,
           l_q_neighbors, l_l_neighbors, q_embed, l_embed, a_embed,
           key_matrix, value_matrix_init,
           W_QQ, W_QL, W_LL, W_LQ, W_GQ, b_GQ, W_GL, b_GL,
           W_kq, b_kq, W_kl, b_kl, W_eq, b_eq, W_el, b_el,
           W_aq, b_aq, W_al, b_al, T_QQ, T_QL, T_LQ, T_LL,
           W_sum, b_sum, W_out, b_out, W_tq, b_tq, W_tl, b_tl):
    f32 = jnp.float32
    qi = q_data[:, 1:_S - 1].T.reshape(-1).astype(jnp.int32)
    li = l_data[:, 1:_S - 1].T.reshape(-1).astype(jnp.int32)
    aux = jnp.stack([
        (d_data[:, 1:_S - 1].T == 0).astype(f32),
        a_data[:, 1:_S - 1].T.astype(f32),
        (d_data[:, 0:_S - 2].T == 0).astype(f32),
        jnp.zeros((_T, _B), f32),
    ], axis=-1).reshape(_PAIRS, 4)

    nbq = jnp.concatenate([q_q_neighbors, q_l_neighbors], axis=1)
    nbl = jnp.concatenate([l_l_neighbors, l_q_neighbors], axis=1)
    qle, x4 = _sc_gather_fn()(qi, li, nbq, nbl, q_embed, l_embed)

    z32 = jnp.zeros((_E, _E), f32)
    Wnb = jnp.concatenate([
        jnp.concatenate([W_QQ, z32], axis=1),
        jnp.concatenate([W_QL, z32], axis=1),
        jnp.concatenate([z32, W_LL], axis=1),
        jnp.concatenate([z32, W_LQ], axis=1),
    ], axis=0)
    bd = lambda a, b: jnp.concatenate([
        jnp.concatenate([a, z32], axis=1),
        jnp.concatenate([z32, b], axis=1)], axis=0)
    cat1 = lambda a, b: jnp.concatenate([a, b], axis=1)
    Wg2 = bd(W_GQ, W_GL)
    bg2 = cat1(b_GQ.reshape(1, _E), b_GL.reshape(1, _E))
    Wk2 = bd(W_kq, W_kl)
    bk2 = cat1(b_kq.reshape(1, _E), b_kl.reshape(1, _E))
    WeaQ = jnp.concatenate([cat1(W_eq[:_E], W_aq[:_E]),
                            cat1(W_eq[_E:], W_aq[_E:])], axis=0)
    beaQ = cat1(b_eq.reshape(1, _E), b_aq.reshape(1, _E))
    WeaL = cat1(W_el, W_al)
    beaL = cat1(b_el.reshape(1, _E), b_al.reshape(1, _E))
    Tcat = jnp.concatenate([T_QQ, T_QL, T_LQ, T_LL], axis=1)

    (out,) = _tc_scan_fn()(
        x4, qle, aux, a_embed,
        Wnb, Wg2, bg2, Wk2, bk2, key_matrix.T,
        WeaQ, beaQ, WeaL, beaL,
        Tcat.astype(jnp.bfloat16), W_sum[_E:], b_sum.reshape(1, -1), W_sum[:_E],
        W_out.T, W_tq.T, W_tl.T, value_matrix_init,
        b_out, b_tq, b_tl)

    return jnp.swapaxes(out.reshape(_T, _B, 2), 0, 1)


# speedup vs baseline: 1.3031x; 1.0378x over previous
import functools

import jax
import jax.numpy as jnp
from jax import lax
from jax.experimental import pallas as pl
from jax.experimental.pallas import tpu as pltpu
from jax.experimental.pallas import tpu_sc as plsc

_B = 64
_S = 50
_T = _S - 2
_E = 32
_C = 32
_NB = 16
_NC, _NS = 2, 16
_NW = _NC * _NS
_PAIRS = _B * _T
_PPW = _PAIRS // _NW
_CHUNK = 128
_NCHUNK = _PPW * _NB // _CHUNK
_RC = 384
_NRC = _PAIRS // _RC


def _sc_body(qi_hbm, li_hbm, nbq_hbm, nbl_hbm, qe_hbm, le_hbm,
             out_qle, out_x4,
             idx_v, nbr_v, flat_a, flat_b, rows_v, erow_v, ebuf_v, sem, esem):
    wid = lax.axis_index("s") * _NC + lax.axis_index("c")
    base = wid * _PPW
    nflat = _PPW * _NB
    nch = nflat // _CHUNK

    def side(i_hbm, e_hbm, nb_hbm, emb_a, emb_b, qle_col, x4_col):
        pltpu.sync_copy(i_hbm.at[pl.ds(base, _PPW)], idx_v)
        ecp = pltpu.async_copy(e_hbm.at[idx_v], ebuf_v, esem)
        pltpu.async_copy(nb_hbm.at[idx_v], nbr_v, sem).wait()

        def flat_body(p, carry):
            flat_a[pl.ds(p * _NB, _NB)] = nbr_v[p, pl.ds(0, _NB)]
            flat_b[pl.ds(p * _NB, _NB)] = nbr_v[p, pl.ds(_NB, _NB)]
            return carry

        lax.fori_loop(0, _PPW, flat_body, 0)
        ecp.wait()
        pltpu.sync_copy(ebuf_v, out_qle.at[pl.ds(base, _PPW), pl.ds(qle_col, _E)])

        for half, (flat, emb) in enumerate(((flat_a, emb_a), (flat_b, emb_b))):
            copies = [
                pltpu.async_copy(emb.at[flat.at[pl.ds(c * _CHUNK, _CHUNK)]],
                                 rows_v.at[pl.ds(c * _CHUNK, _CHUNK)], sem)
                for c in range(nch)
            ]
            for cp in copies:
                cp.wait()

            col0 = half * _E

            def acc_body(p, carry):
                lo = rows_v[p * _NB, pl.ds(0, 16)]
                hi = rows_v[p * _NB, pl.ds(16, 16)]
                for j in range(1, _NB):
                    lo = lo + rows_v[p * _NB + j, pl.ds(0, 16)]
                    hi = hi + rows_v[p * _NB + j, pl.ds(16, 16)]
                nv = nbr_v[p, pl.ds(half * _NB, _NB)]
                cnt = jnp.int32(0)
                for j in range(_NB):
                    cnt = cnt + jnp.where(nv[j] != 0, 1, 0).astype(jnp.int32)
                inv = jnp.float32(1.0)
                for k in range(2, _NB + 1):
                    inv = jnp.where(cnt == k, jnp.float32(1.0 / k), inv)
                erow_v[p, pl.ds(col0, 16)] = lo * inv
                erow_v[p, pl.ds(col0 + 16, 16)] = hi * inv
                return carry

            lax.fori_loop(0, _PPW, acc_body, 0)

        pltpu.sync_copy(
            erow_v, out_x4.at[pl.ds(base, _PPW), pl.ds(x4_col, 2 * _E)])

    side(qi_hbm, qe_hbm, nbq_hbm, qe_hbm, le_hbm, 0, 0)
    side(li_hbm, le_hbm, nbl_hbm, le_hbm, qe_hbm, _E, 2 * _E)


@functools.cache
def _sc_gather_fn():
    return pl.kernel(
        _sc_body,
        out_type=[jax.ShapeDtypeStruct((_PAIRS, 2 * _E), jnp.float32),
                  jax.ShapeDtypeStruct((_PAIRS, 4 * _E), jnp.float32)],
        mesh=plsc.VectorSubcoreMesh(core_axis_name="c", subcore_axis_name="s",
                                    num_cores=_NC, num_subcores=_NS),
        compiler_params=pltpu.CompilerParams(use_tc_tiling_on_sc=False),
        scratch_types=[
            pltpu.VMEM((_PPW,), jnp.int32),
            pltpu.VMEM((_PPW, 2 * _NB), jnp.int32),
            pltpu.VMEM((_PPW * _NB,), jnp.int32),
            pltpu.VMEM((_PPW * _NB,), jnp.int32),
            pltpu.VMEM((_PPW * _NB, _E), jnp.float32),
            pltpu.VMEM((_PPW, 2 * _E), jnp.float32),
            pltpu.VMEM((_PPW, _E), jnp.float32),
            pltpu.SemaphoreType.DMA,
            pltpu.SemaphoreType.DMA,
        ],
    )


def _sig(x):
    return 1.0 / (1.0 + jnp.exp(-x))


def _tc_body(x4_r, qle_r, aux_r, aemb_r,
             wnb_r, wg2_r, bg2_r, wk2_r, bk2_r, keyt_r,
             weaq_r, beaq_r, weal_r, beal_r,
             tcat_r, wsq_r, bsum_r, wsr_r, wout_r, wtq_r, wtl_r, vmi_r,
             bout_r, btq_r, btl_r,
             out_r,
             w_s, er_s, ad_s, ps_s, rd_s):
    Wnb = wnb_r[...]; Wg2 = wg2_r[...]; bg2 = bg2_r[...]
    Wk2 = wk2_r[...]; bk2 = bk2_r[...]; keyT = keyt_r[...]
    WeaQ = weaq_r[...]; beaQ = beaq_r[...]
    WeaL = weal_r[...]; beaL = beal_r[...]
    Tcat = tcat_r[...]; Wsq = wsq_r[...]; bsum = bsum_r[...]; Wsr = wsr_r[...]
    wout = wout_r[...]; wtq = wtq_r[...]; wtl = wtl_r[...]
    bout = bout_r[0]; btq = btq_r[0]; btl = btl_r[0]
    a0 = aemb_r[0:1, :]
    a1 = aemb_r[1:2, :]

    def phase_a(i, carry):
        b0 = i * _RC
        x4 = x4_r[pl.ds(b0, _RC), :]
        qle = qle_r[pl.ds(b0, _RC), :]
        aux = aux_r[pl.ds(b0, _RC), :]
        iqc = aux[:, 0:1] > 0.5
        am = aux[:, 1:2]
        a_t = am * a1 + (1.0 - am) * a0
        y = jnp.dot(x4, Wnb)
        z = jnp.tanh(jnp.dot(qle + y, Wg2) + bg2)
        u = jnp.dot(z, Wk2) + bk2
        usel = jnp.where(iqc, u[:, :_E], u[:, _E:])
        klog = jnp.dot(usel, keyT)
        mx = jnp.max(klog, axis=1, keepdims=True)
        ex = jnp.exp(klog - mx)
        w = ex / jnp.sum(ex, axis=1, keepdims=True)
        q_t = z[:, :_E]
        eaq = jnp.dot(jnp.concatenate([q_t, a_t], axis=1), WeaQ) + beaQ
        eal = jnp.dot(z[:, _E:], WeaL) + beaL
        er = jnp.where(iqc, _sig(eaq[:, :_E]), _sig(eal[:, :_E]))
        ad = jnp.where(iqc, jnp.tanh(eaq[:, _E:]), jnp.tanh(eal[:, _E:]))
        ps = jnp.dot(q_t, Wsq) + bsum
        w_s[pl.ds(b0, _RC), :] = w
        er_s[pl.ds(b0, _RC), :] = er
        ad_s[pl.ds(b0, _RC), :] = ad
        ps_s[pl.ds(b0, _RC), :] = ps
        return carry

    lax.fori_loop(0, _NRC, phase_a, 0, unroll=False)

    def phase_b(t, vm):
        b0 = t * _B
        aux = aux_r[pl.ds(b0, _B), :]
        iq3 = aux[:, 0:1][:, :, None] > 0.5
        pq3 = aux[:, 2:3][:, :, None] > 0.5
        vm2 = vm.reshape(_B * _C, _E).astype(jnp.bfloat16)
        prod = jnp.dot(vm2, Tcat, preferred_element_type=jnp.float32)
        vqq = prod[:, 0:_E].reshape(_B, _C, _E)
        vql = prod[:, _E:2 * _E].reshape(_B, _C, _E)
        vlq = prod[:, 2 * _E:3 * _E].reshape(_B, _C, _E)
        vll = prod[:, 3 * _E:].reshape(_B, _C, _E)
        vm = jnp.where(pq3, jnp.where(iq3, vqq, vql), jnp.where(iq3, vlq, vll))
        w3 = w_s[pl.ds(b0, _B), :][:, :, None]
        read = jnp.sum(w3 * vm, axis=1)
        rd_s[pl.ds(b0, _B), :] = read
        er = er_s[pl.ds(b0, _B), :]
        ad = ad_s[pl.ds(b0, _B), :]
        return vm + w3 * (ad[:, None, :] - vm * er[:, None, :])

    lax.fori_loop(0, _T, phase_b,
                  jnp.broadcast_to(vmi_r[...], (_B, _C, _E)), unroll=2)

    def phase_c(i, carry):
        b0 = i * _RC
        read = rd_s[pl.ds(b0, _RC), :]
        aux = aux_r[pl.ds(b0, _RC), :]
        summ = jnp.tanh(jnp.dot(read, Wsr) + ps_s[pl.ds(b0, _RC), :])
        pred = _sig(jnp.sum(summ * wout, axis=1, keepdims=True) + bout)
        ptq = jnp.sum(read * wtq, axis=1, keepdims=True) + btq
        ptl = jnp.sum(read * wtl, axis=1, keepdims=True) + btl
        ptype = _sig(jnp.where(aux[:, 0:1] > 0.5, ptq, ptl))
        out_r[pl.ds(b0, _RC), :] = jnp.concatenate([pred, ptype], axis=1)
        return carry

    lax.fori_loop(0, _NRC, phase_c, 0, unroll=False)


_N_VMEM_IN = 22


@functools.cache
def _tc_scan_fn():
    return pl.pallas_call(
        _tc_body,
        out_shape=[jax.ShapeDtypeStruct((_PAIRS, 2), jnp.float32)],
        in_specs=[pl.BlockSpec(memory_space=pltpu.VMEM)] * _N_VMEM_IN
        + [pl.BlockSpec(memory_space=pltpu.SMEM)] * 3,
        scratch_shapes=[
            pltpu.VMEM((_PAIRS, _E), jnp.float32),
            pltpu.VMEM((_PAIRS, _E), jnp.float32),
            pltpu.VMEM((_PAIRS, _E), jnp.float32),
            pltpu.VMEM((_PAIRS, 2 * _E), jnp.float32),
            pltpu.VMEM((_PAIRS, _E), jnp.float32),
        ],
    )


def kernel(q_data, a_data, l_data, d_data, q_q_neighbors, q_l_neighbors,
           l_q_neighbors, l_l_neighbors, q_embed, l_embed, a_embed,
           key_matrix, value_matrix_init,
           W_QQ, W_QL, W_LL, W_LQ, W_GQ, b_GQ, W_GL, b_GL,
           W_kq, b_kq, W_kl, b_kl, W_eq, b_eq, W_el, b_el,
           W_aq, b_aq, W_al, b_al, T_QQ, T_QL, T_LQ, T_LL,
           W_sum, b_sum, W_out, b_out, W_tq, b_tq, W_tl, b_tl):
    f32 = jnp.float32
    qi = q_data[:, 1:_S - 1].T.reshape(-1).astype(jnp.int32)
    li = l_data[:, 1:_S - 1].T.reshape(-1).astype(jnp.int32)
    aux = jnp.stack([
        (d_data[:, 1:_S - 1].T == 0).astype(f32),
        a_data[:, 1:_S - 1].T.astype(f32),
        (d_data[:, 0:_S - 2].T == 0).astype(f32),
        jnp.zeros((_T, _B), f32),
    ], axis=-1).reshape(_PAIRS, 4)

    nbq = jnp.concatenate([q_q_neighbors, q_l_neighbors], axis=1)
    nbl = jnp.concatenate([l_l_neighbors, l_q_neighbors], axis=1)
    qle, x4 = _sc_gather_fn()(qi, li, nbq, nbl, q_embed, l_embed)

    z32 = jnp.zeros((_E, _E), f32)
    Wnb = jnp.concatenate([
        jnp.concatenate([W_QQ, z32], axis=1),
        jnp.concatenate([W_QL, z32], axis=1),
        jnp.concatenate([z32, W_LL], axis=1),
        jnp.concatenate([z32, W_LQ], axis=1),
    ], axis=0)
    bd = lambda a, b: jnp.concatenate([
        jnp.concatenate([a, z32], axis=1),
        jnp.concatenate([z32, b], axis=1)], axis=0)
    cat1 = lambda a, b: jnp.concatenate([a, b], axis=1)
    Wg2 = bd(W_GQ, W_GL)
    bg2 = cat1(b_GQ.reshape(1, _E), b_GL.reshape(1, _E))
    Wk2 = bd(W_kq, W_kl)
    bk2 = cat1(b_kq.reshape(1, _E), b_kl.reshape(1, _E))
    WeaQ = jnp.concatenate([cat1(W_eq[:_E], W_aq[:_E]),
                            cat1(W_eq[_E:], W_aq[_E:])], axis=0)
    beaQ = cat1(b_eq.reshape(1, _E), b_aq.reshape(1, _E))
    WeaL = cat1(W_el, W_al)
    beaL = cat1(b_el.reshape(1, _E), b_al.reshape(1, _E))
    Tcat = jnp.concatenate([T_QQ, T_QL, T_LQ, T_LL], axis=1)

    (out,) = _tc_scan_fn()(
        x4, qle, aux, a_embed,
        Wnb, Wg2, bg2, Wk2, bk2, key_matrix.T,
        WeaQ, beaQ, WeaL, beaL,
        Tcat.astype(jnp.bfloat16), W_sum[_E:], b_sum.reshape(1, -1), W_sum[:_E],
        W_out.T, W_tq.T, W_tl.T, value_matrix_init,
        b_out, b_tq, b_tl)

    return jnp.swapaxes(out.reshape(_T, _B, 2), 0, 1)


# speedup vs baseline: 1.3215x; 1.0141x over previous
import functools

import jax
import jax.numpy as jnp
from jax import lax
from jax.experimental import pallas as pl
from jax.experimental.pallas import tpu as pltpu
from jax.experimental.pallas import tpu_sc as plsc

_B = 64
_S = 50
_T = _S - 2
_E = 32
_C = 32
_NB = 16
_NC, _NS = 2, 16
_NW = _NC * _NS
_PAIRS = _B * _T
_PPW = _PAIRS // _NW
_CHUNK = 128
_NCHUNK = _PPW * _NB // _CHUNK
_RC = 384
_NRC = _PAIRS // _RC


def _sc_body(qi_hbm, li_hbm, nbq_hbm, nbl_hbm, qe_hbm, le_hbm,
             out_qle, out_x4,
             idx_v, nbr_v, flat_a, flat_b, rows_v, erow_v, ebuf_v, sem, esem):
    wid = lax.axis_index("s") * _NC + lax.axis_index("c")
    base = wid * _PPW
    nflat = _PPW * _NB
    nch = nflat // _CHUNK

    def side(i_hbm, e_hbm, nb_hbm, emb_a, emb_b, qle_col, x4_col):
        pltpu.sync_copy(i_hbm.at[pl.ds(base, _PPW)], idx_v)
        ecp = pltpu.async_copy(e_hbm.at[idx_v], ebuf_v, esem)
        pltpu.async_copy(nb_hbm.at[idx_v], nbr_v, sem).wait()

        def flat_body(p, carry):
            flat_a[pl.ds(p * _NB, _NB)] = nbr_v[p, pl.ds(0, _NB)]
            flat_b[pl.ds(p * _NB, _NB)] = nbr_v[p, pl.ds(_NB, _NB)]
            return carry

        lax.fori_loop(0, _PPW, flat_body, 0)
        ecp.wait()
        pltpu.sync_copy(ebuf_v, out_qle.at[pl.ds(base, _PPW), pl.ds(qle_col, _E)])

        for half, (flat, emb) in enumerate(((flat_a, emb_a), (flat_b, emb_b))):
            copies = [
                pltpu.async_copy(emb.at[flat.at[pl.ds(c * _CHUNK, _CHUNK)]],
                                 rows_v.at[pl.ds(c * _CHUNK, _CHUNK)], sem)
                for c in range(nch)
            ]
            for cp in copies:
                cp.wait()

            col0 = half * _E

            def acc_body(p, carry):
                lo = rows_v[p * _NB, pl.ds(0, 16)]
                hi = rows_v[p * _NB, pl.ds(16, 16)]
                for j in range(1, _NB):
                    lo = lo + rows_v[p * _NB + j, pl.ds(0, 16)]
                    hi = hi + rows_v[p * _NB + j, pl.ds(16, 16)]
                nv = nbr_v[p, pl.ds(half * _NB, _NB)]
                cnt = jnp.int32(0)
                for j in range(_NB):
                    cnt = cnt + jnp.where(nv[j] != 0, 1, 0).astype(jnp.int32)
                inv = jnp.float32(1.0)
                for k in range(2, _NB + 1):
                    inv = jnp.where(cnt == k, jnp.float32(1.0 / k), inv)
                erow_v[p, pl.ds(col0, 16)] = lo * inv
                erow_v[p, pl.ds(col0 + 16, 16)] = hi * inv
                return carry

            lax.fori_loop(0, _PPW, acc_body, 0)

        pltpu.sync_copy(
            erow_v, out_x4.at[pl.ds(base, _PPW), pl.ds(x4_col, 2 * _E)])

    side(qi_hbm, qe_hbm, nbq_hbm, qe_hbm, le_hbm, 0, 0)
    side(li_hbm, le_hbm, nbl_hbm, le_hbm, qe_hbm, _E, 2 * _E)


@functools.cache
def _sc_gather_fn():
    return pl.kernel(
        _sc_body,
        out_type=[jax.ShapeDtypeStruct((_PAIRS, 2 * _E), jnp.float32),
                  jax.ShapeDtypeStruct((_PAIRS, 4 * _E), jnp.float32)],
        mesh=plsc.VectorSubcoreMesh(core_axis_name="c", subcore_axis_name="s",
                                    num_cores=_NC, num_subcores=_NS),
        compiler_params=pltpu.CompilerParams(use_tc_tiling_on_sc=False),
        scratch_types=[
            pltpu.VMEM((_PPW,), jnp.int32),
            pltpu.VMEM((_PPW, 2 * _NB), jnp.int32),
            pltpu.VMEM((_PPW * _NB,), jnp.int32),
            pltpu.VMEM((_PPW * _NB,), jnp.int32),
            pltpu.VMEM((_PPW * _NB, _E), jnp.float32),
            pltpu.VMEM((_PPW, 2 * _E), jnp.float32),
            pltpu.VMEM((_PPW, _E), jnp.float32),
            pltpu.SemaphoreType.DMA,
            pltpu.SemaphoreType.DMA,
        ],
    )


def _sig(x):
    return 1.0 / (1.0 + jnp.exp(-x))


def _tc_body(x4_r, qle_r, aux_r, aemb_r,
             wnb_r, wg2_r, bg2_r, wk2_r, bk2_r, keyt_r,
             weaq_r, beaq_r, weal_r, beal_r,
             tcat_r, wsq_r, bsum_r, wsr_r, wout_r, wtq_r, wtl_r, vmi_r,
             bout_r, btq_r, btl_r,
             out_r,
             w_s, er_s, ad_s, ps_s, rd_s):
    Wnb = wnb_r[...]; Wg2 = wg2_r[...]; bg2 = bg2_r[...]
    Wk2 = wk2_r[...]; bk2 = bk2_r[...]; keyT = keyt_r[...]
    WeaQ = weaq_r[...]; beaQ = beaq_r[...]
    WeaL = weal_r[...]; beaL = beal_r[...]
    Tcat = tcat_r[...]; Wsq = wsq_r[...]; bsum = bsum_r[...]; Wsr = wsr_r[...]
    wout = wout_r[...]; wtq = wtq_r[...]; wtl = wtl_r[...]
    bout = bout_r[0]; btq = btq_r[0]; btl = btl_r[0]
    a0 = aemb_r[0:1, :]
    a1 = aemb_r[1:2, :]

    def phase_a(i, carry):
        b0 = i * _RC
        x4 = x4_r[pl.ds(b0, _RC), :]
        qle = qle_r[pl.ds(b0, _RC), :]
        aux = aux_r[pl.ds(b0, _RC), :]
        iqc = aux[:, 0:1] > 0.5
        am = aux[:, 1:2]
        a_t = am * a1 + (1.0 - am) * a0
        y = jnp.dot(x4, Wnb)
        z = jnp.tanh(jnp.dot(qle + y, Wg2) + bg2)
        u = jnp.dot(z, Wk2) + bk2
        usel = jnp.where(iqc, u[:, :_E], u[:, _E:])
        klog = jnp.dot(usel, keyT)
        mx = jnp.max(klog, axis=1, keepdims=True)
        ex = jnp.exp(klog - mx)
        w = ex / jnp.sum(ex, axis=1, keepdims=True)
        q_t = z[:, :_E]
        eaq = jnp.dot(jnp.concatenate([q_t, a_t], axis=1), WeaQ) + beaQ
        eal = jnp.dot(z[:, _E:], WeaL) + beaL
        er = jnp.where(iqc, _sig(eaq[:, :_E]), _sig(eal[:, :_E]))
        ad = jnp.where(iqc, jnp.tanh(eaq[:, _E:]), jnp.tanh(eal[:, _E:]))
        ps = jnp.dot(q_t, Wsq) + bsum
        w_s[pl.ds(b0, _RC), :] = w
        er_s[pl.ds(b0, _RC), :] = er
        ad_s[pl.ds(b0, _RC), :] = ad
        ps_s[pl.ds(b0, _RC), :] = ps
        return carry

    lax.fori_loop(0, _NRC, phase_a, 0, unroll=False)

    def phase_b(t, vm):
        b0 = t * _B
        aux = aux_r[pl.ds(b0, _B), :]
        iq3 = aux[:, 0:1][:, :, None] > 0.5
        pq3 = aux[:, 2:3][:, :, None] > 0.5
        vm2 = vm.reshape(_B * _C, _E).astype(jnp.bfloat16)
        prod = jnp.dot(vm2, Tcat, preferred_element_type=jnp.float32)
        vqq = prod[:, 0:_E].reshape(_B, _C, _E)
        vql = prod[:, _E:2 * _E].reshape(_B, _C, _E)
        vlq = prod[:, 2 * _E:3 * _E].reshape(_B, _C, _E)
        vll = prod[:, 3 * _E:].reshape(_B, _C, _E)
        vm = jnp.where(pq3, jnp.where(iq3, vqq, vql), jnp.where(iq3, vlq, vll))
        w3 = w_s[pl.ds(b0, _B), :][:, :, None]
        read = jnp.sum(w3 * vm, axis=1)
        rd_s[pl.ds(b0, _B), :] = read
        er = er_s[pl.ds(b0, _B), :]
        ad = ad_s[pl.ds(b0, _B), :]
        return vm + w3 * (ad[:, None, :] - vm * er[:, None, :])

    lax.fori_loop(0, _T, phase_b,
                  jnp.broadcast_to(vmi_r[...], (_B, _C, _E)), unroll=4)

    def phase_c(i, carry):
        b0 = i * _RC
        read = rd_s[pl.ds(b0, _RC), :]
        aux = aux_r[pl.ds(b0, _RC), :]
        summ = jnp.tanh(jnp.dot(read, Wsr) + ps_s[pl.ds(b0, _RC), :])
        pred = _sig(jnp.sum(summ * wout, axis=1, keepdims=True) + bout)
        ptq = jnp.sum(read * wtq, axis=1, keepdims=True) + btq
        ptl = jnp.sum(read * wtl, axis=1, keepdims=True) + btl
        ptype = _sig(jnp.where(aux[:, 0:1] > 0.5, ptq, ptl))
        out_r[pl.ds(b0, _RC), :] = jnp.concatenate([pred, ptype], axis=1)
        return carry

    lax.fori_loop(0, _NRC, phase_c, 0, unroll=False)


_N_VMEM_IN = 22


@functools.cache
def _tc_scan_fn():
    return pl.pallas_call(
        _tc_body,
        out_shape=[jax.ShapeDtypeStruct((_PAIRS, 2), jnp.float32)],
        in_specs=[pl.BlockSpec(memory_space=pltpu.VMEM)] * _N_VMEM_IN
        + [pl.BlockSpec(memory_space=pltpu.SMEM)] * 3,
        scratch_shapes=[
            pltpu.VMEM((_PAIRS, _E), jnp.float32),
            pltpu.VMEM((_PAIRS, _E), jnp.float32),
            pltpu.VMEM((_PAIRS, _E), jnp.float32),
            pltpu.VMEM((_PAIRS, 2 * _E), jnp.float32),
            pltpu.VMEM((_PAIRS, _E), jnp.float32),
        ],
    )


def kernel(q_data, a_data, l_data, d_data, q_q_neighbors, q_l_neighbors,
           l_q_neighbors, l_l_neighbors, q_embed, l_embed, a_embed,
           key_matrix, value_matrix_init,
           W_QQ, W_QL, W_LL, W_LQ, W_GQ, b_GQ, W_GL, b_GL,
           W_kq, b_kq, W_kl, b_kl, W_eq, b_eq, W_el, b_el,
           W_aq, b_aq, W_al, b_al, T_QQ, T_QL, T_LQ, T_LL,
           W_sum, b_sum, W_out, b_out, W_tq, b_tq, W_tl, b_tl):
    f32 = jnp.float32
    qi = q_data[:, 1:_S - 1].T.reshape(-1).astype(jnp.int32)
    li = l_data[:, 1:_S - 1].T.reshape(-1).astype(jnp.int32)
    aux = jnp.stack([
        (d_data[:, 1:_S - 1].T == 0).astype(f32),
        a_data[:, 1:_S - 1].T.astype(f32),
        (d_data[:, 0:_S - 2].T == 0).astype(f32),
        jnp.zeros((_T, _B), f32),
    ], axis=-1).reshape(_PAIRS, 4)

    nbq = jnp.concatenate([q_q_neighbors, q_l_neighbors], axis=1)
    nbl = jnp.concatenate([l_l_neighbors, l_q_neighbors], axis=1)
    qle, x4 = _sc_gather_fn()(qi, li, nbq, nbl, q_embed, l_embed)

    z32 = jnp.zeros((_E, _E), f32)
    Wnb = jnp.concatenate([
        jnp.concatenate([W_QQ, z32], axis=1),
        jnp.concatenate([W_QL, z32], axis=1),
        jnp.concatenate([z32, W_LL], axis=1),
        jnp.concatenate([z32, W_LQ], axis=1),
    ], axis=0)
    bd = lambda a, b: jnp.concatenate([
        jnp.concatenate([a, z32], axis=1),
        jnp.concatenate([z32, b], axis=1)], axis=0)
    cat1 = lambda a, b: jnp.concatenate([a, b], axis=1)
    Wg2 = bd(W_GQ, W_GL)
    bg2 = cat1(b_GQ.reshape(1, _E), b_GL.reshape(1, _E))
    Wk2 = bd(W_kq, W_kl)
    bk2 = cat1(b_kq.reshape(1, _E), b_kl.reshape(1, _E))
    WeaQ = jnp.concatenate([cat1(W_eq[:_E], W_aq[:_E]),
                            cat1(W_eq[_E:], W_aq[_E:])], axis=0)
    beaQ = cat1(b_eq.reshape(1, _E), b_aq.reshape(1, _E))
    WeaL = cat1(W_el, W_al)
    beaL = cat1(b_el.reshape(1, _E), b_al.reshape(1, _E))
    Tcat = jnp.concatenate([T_QQ, T_QL, T_LQ, T_LL], axis=1)

    (out,) = _tc_scan_fn()(
        x4, qle, aux, a_embed,
        Wnb, Wg2, bg2, Wk2, bk2, key_matrix.T,
        WeaQ, beaQ, WeaL, beaL,
        Tcat.astype(jnp.bfloat16), W_sum[_E:], b_sum.reshape(1, -1), W_sum[:_E],
        W_out.T, W_tq.T, W_tl.T, value_matrix_init,
        b_out, b_tq, b_tl)

    return jnp.swapaxes(out.reshape(_T, _B, 2), 0, 1)
